# baseline (speedup 1.0000x reference)
import functools

import jax
import jax.numpy as jnp
from jax import lax
from jax.experimental import pallas as pl
from jax.experimental.pallas import tpu as pltpu
from jax.experimental.pallas import tpu_sc as plsc

N = 10000
E = 320000
F = 128
NUM_GRAPHS = 8 * 8

NC = 2
NS = 16
NW = NC * NS
CHUNK = 96
NCHUNK = 105
EPT = NCHUNK * CHUNK
E_PAD = NW * EPT
NPAD = 10240
ROWS_PT = NPAD // NS


def _sc_scatter_kernel(t_hbm, idx_hbm, zero_hbm, out_hbm, acc, idxc, dstc,
                       rows, gsem, isem, ssem0, ssem1, ssem2):
    cid = lax.axis_index("c")
    sid = lax.axis_index("s")
    wid = sid * NC + cid
    ssems = (ssem0, ssem1, ssem2)

    pltpu.sync_copy(zero_hbm, acc.at[pl.ds(sid * ROWS_PT, ROWS_PT)])
    plsc.subcore_barrier()

    def start_idx(i, ib):
        pltpu.async_copy(idx_hbm.at[wid, i], idxc.at[ib], isem)

    def wait_idx(i, ib):
        pltpu.make_async_copy(idx_hbm.at[wid, i], idxc.at[ib], isem).wait()

    def start_gather(i, ib, b):
        pltpu.async_copy(t_hbm.at[idxc.at[ib, 0]], rows.at[b], gsem)

    def wait_gather(b):
        pltpu.make_async_copy(t_hbm.at[idxc.at[0, 0]], rows.at[b], gsem).wait()

    def scale(ib, b):
        @plsc.parallel_loop(0, CHUNK, step=1, unroll=8)
        def _scale(e):
            a = plsc.bitcast(
                plsc.load_gather(idxc.at[ib, 2], [jnp.full((16,), e, jnp.int32)]),
                jnp.float32)
            for j in range(F // 16):
                rows[b, e, pl.ds(j * 16, 16)] = rows[b, e, pl.ds(j * 16, 16)] * a

    def start_scatter(b):
        pltpu.async_copy(rows.at[b], acc.at[dstc.at[b]], ssems[b], add=True)

    def wait_scatter(b):
        pltpu.make_async_copy(rows.at[b], acc.at[dstc.at[b]], ssems[b]).wait()

    def handle(ch, ib, b, first, last):
        wait_gather(b)
        if not last:
            wait_idx(ch + 1, 1 - ib)
        if not last:
            start_gather(ch + 1, 1 - ib, (b + 1) % 3)
        scale(ib, b)
        pltpu.sync_copy(rows.at[b], acc.at[idxc.at[ib, 1]], add=True)
        if not last:
            start_idx(jnp.minimum(ch + 2, NCHUNK - 1), ib)

    pltpu.sync_copy(idx_hbm.at[wid, 0], idxc.at[0])
    start_idx(1, 1)
    start_gather(0, 0, 0)
    handle(0, 0, 0, True, False)
    handle(1, 1, 1, True, False)

    def pipe_body(i, carry):
        for k in range(6):
            ch = 2 + 6 * i + k
            handle(ch, k % 2, (2 + k) % 3, False, False)
        return carry
    lax.fori_loop(0, (NCHUNK - 3) // 6, pipe_body, 0)
    wait_idx(NCHUNK - 1, (NCHUNK - 1) % 2)
    handle(NCHUNK - 1, (NCHUNK - 1) % 2, (NCHUNK - 1) % 3, False, True)

    plsc.subcore_barrier()
    sl = pl.ds(sid * ROWS_PT, ROWS_PT)
    pltpu.sync_copy(acc.at[sl], out_hbm.at[cid, sl])


_sc_scatter = functools.partial(
    pl.kernel,
    mesh=plsc.VectorSubcoreMesh(core_axis_name="c", subcore_axis_name="s"),
    out_type=jax.ShapeDtypeStruct((NC, NPAD, F), jnp.float32),
    scratch_types=[
        pltpu.VMEM_SHARED((NPAD, F), jnp.float32),
        pltpu.VMEM((2, 3, CHUNK), jnp.int32),
        pltpu.VMEM((3, CHUNK), jnp.int32),
        pltpu.VMEM((3, CHUNK, F), jnp.float32),
        pltpu.SemaphoreType.DMA,
        pltpu.SemaphoreType.DMA,
        pltpu.SemaphoreType.DMA,
        pltpu.SemaphoreType.DMA,
        pltpu.SemaphoreType.DMA,
    ],
    compiler_params=pltpu.CompilerParams(needs_layout_passes=False),
)(_sc_scatter_kernel)


def _msg_body(h_ref, w1_ref, b1_ref, w2_ref, b2_ref, t_ref):
    a = jnp.maximum(
        jnp.dot(h_ref[...], w1_ref[...], preferred_element_type=jnp.float32)
        + b1_ref[...], 0.0)
    t_ref[...] = jnp.dot(a, w2_ref[...], preferred_element_type=jnp.float32) \
        + b2_ref[...]


def _msg_tc(h, w1, b1, w2, b2):
    return pl.pallas_call(
        _msg_body,
        out_shape=jax.ShapeDtypeStruct((N, F), jnp.float32),
    )(h, w1, b1, w2, b2)


def _update_h(h_ref, g0_ref, g1_ref, ew_ref, w1h_ref, w1a_ref, b1_ref,
              w2_ref, b2_ref, gam_ref, bet_ref):
    agg = (g0_ref[:N, :] + g1_ref[:N, :]) * ew_ref[...]
    a = jnp.maximum(
        jnp.dot(h_ref[...], w1h_ref[...], preferred_element_type=jnp.float32)
        + jnp.dot(agg, w1a_ref[...], preferred_element_type=jnp.float32)
        + b1_ref[...], 0.0)
    u = jnp.dot(a, w2_ref[...], preferred_element_type=jnp.float32) + b2_ref[...]
    mean = jnp.mean(u, axis=0, keepdims=True)
    var = jnp.mean((u - mean) * (u - mean), axis=0, keepdims=True)
    hn = gam_ref[...] * (u - mean) / jnp.sqrt(var + 1e-5) + bet_ref[...]
    return jnp.maximum(hn, 0.0)


def _update_msg_body(h_ref, g0_ref, g1_ref, ew_ref, w1h_ref, w1a_ref, b1_ref,
                     w2_ref, b2_ref, gam_ref, bet_ref,
                     mw1_ref, mb1_ref, mw2_ref, mb2_ref, h_out, t_out):
    hn = _update_h(h_ref, g0_ref, g1_ref, ew_ref, w1h_ref, w1a_ref, b1_ref,
                   w2_ref, b2_ref, gam_ref, bet_ref)
    h_out[...] = hn
    a = jnp.maximum(
        jnp.dot(hn, mw1_ref[...], preferred_element_type=jnp.float32)
        + mb1_ref[...], 0.0)
    t_out[...] = jnp.dot(a, mw2_ref[...], preferred_element_type=jnp.float32) \
        + mb2_ref[...]


def _update_msg_tc(h, g0, g1, ew, w1h, w1a, b1, w2, b2, gam, bet,
                   mw1, mb1, mw2, mb2):
    return pl.pallas_call(
        _update_msg_body,
        out_shape=(jax.ShapeDtypeStruct((N, F), jnp.float32),
                   jax.ShapeDtypeStruct((N, F), jnp.float32)),
    )(h, g0, g1, ew, w1h, w1a, b1, w2, b2, gam, bet, mw1, mb1, mw2, mb2)


def _update_pool_body(h_ref, g0_ref, g1_ref, ew_ref, w1h_ref, w1a_ref, b1_ref,
                      w2_ref, b2_ref, gam_ref, bet_ref,
                      batch_ref, lw_ref, lb_ref, out_ref):
    hn = _update_h(h_ref, g0_ref, g1_ref, ew_ref, w1h_ref, w1a_ref, b1_ref,
                   w2_ref, b2_ref, gam_ref, bet_ref)
    gid = lax.broadcasted_iota(jnp.int32, (N, NUM_GRAPHS), 1)
    onehot = jnp.where(batch_ref[...] == gid, 1.0, 0.0).astype(jnp.float32)
    pooled = lax.dot_general(onehot, hn, (((0,), (0,)), ((), ())),
                             preferred_element_type=jnp.float32)
    counts = lax.dot_general(onehot, jnp.ones((N, 1), jnp.float32),
                             (((0,), (0,)), ((), ())),
                             preferred_element_type=jnp.float32)
    pooled = pooled / jnp.maximum(counts, 1.0)
    out_ref[...] = jnp.dot(pooled, lw_ref[...],
                           preferred_element_type=jnp.float32) + lb_ref[...]


def _update_pool_tc(h, g0, g1, ew, w1h, w1a, b1, w2, b2, gam, bet,
                    batch2d, lw, lb):
    return pl.pallas_call(
        _update_pool_body,
        out_shape=jax.ShapeDtypeStruct((NUM_GRAPHS, lw.shape[1]), jnp.float32),
    )(h, g0, g1, ew, w1h, w1a, b1, w2, b2, gam, bet, batch2d, lw, lb)


def kernel(x, edge_index, batch, edge_attr, params):
    extra = E_PAD - E
    src = jnp.concatenate(
        [edge_index[0], jnp.zeros((extra,), jnp.int32)]).reshape(
            NW, NCHUNK, CHUNK)
    dst = jnp.concatenate(
        [edge_index[1], jnp.full((extra,), NPAD - 1, jnp.int32)]).reshape(
            NW, NCHUNK, CHUNK)
    attr_bits = jax.lax.bitcast_convert_type(
        jnp.concatenate([edge_attr.reshape(E), jnp.zeros((extra,), jnp.float32)]
                        ).reshape(NW, NCHUNK, CHUNK), jnp.int32)
    idx_packed = jnp.stack([src, dst, attr_bits], axis=2)
    acc_zeros = jnp.zeros((ROWS_PT, F), jnp.float32)
    batch2d = batch.reshape(N, 1)

    def upd_args(h, g, p):
        return (h, g[0], g[1], p['edge_W'],
                p['upd_W1'][:F], p['upd_W1'][F:],
                p['upd_b1'].reshape(1, F), p['upd_W2'],
                p['upd_b2'].reshape(1, F),
                p['bn_gamma'].reshape(1, F), p['bn_beta'].reshape(1, F))

    p0, p1, p2 = params['layers']
    h = x
    t = _msg_tc(h, p0['msg_W1'], p0['msg_b1'].reshape(1, F),
                p0['msg_W2'], p0['msg_b2'].reshape(1, F))
    for p, pn in ((p0, p1), (p1, p2)):
        g = _sc_scatter(t, idx_packed, acc_zeros)
        h, t = _update_msg_tc(*upd_args(h, g, p),
                              pn['msg_W1'], pn['msg_b1'].reshape(1, F),
                              pn['msg_W2'], pn['msg_b2'].reshape(1, F))
    g = _sc_scatter(t, idx_packed, acc_zeros)
    return _update_pool_tc(*upd_args(h, g, p2), batch2d,
                           params['lin_W'], params['lin_b'].reshape(1, -1))


# speedup vs baseline: 1.0048x; 1.0048x over previous
import functools

import jax
import jax.numpy as jnp
from jax import lax
from jax.experimental import pallas as pl
from jax.experimental.pallas import tpu as pltpu
from jax.experimental.pallas import tpu_sc as plsc

N = 10000
E = 320000
F = 128
NUM_GRAPHS = 8 * 8

NC = 2
NS = 16
NW = NC * NS
CHUNK = 96
NCHUNK = 105
EPT = NCHUNK * CHUNK
E_PAD = NW * EPT
NPAD = 10240
ROWS_PT = NPAD // NS


def _sc_scatter_kernel(t_hbm, idx_hbm, zero_hbm, out_hbm, acc, idxc,
                       rows, gsem, isem):
    cid = lax.axis_index("c")
    sid = lax.axis_index("s")
    wid = sid * NC + cid

    pltpu.sync_copy(zero_hbm, acc.at[pl.ds(sid * ROWS_PT, ROWS_PT)])
    plsc.subcore_barrier()

    def start_idx(i, ib):
        pltpu.async_copy(idx_hbm.at[wid, i], idxc.at[ib], isem)

    def wait_idx(i, ib):
        pltpu.make_async_copy(idx_hbm.at[wid, i], idxc.at[ib], isem).wait()

    def start_gather(b):
        pltpu.async_copy(t_hbm.at[idxc.at[b, 0]], rows.at[b], gsem)

    def wait_gather(b):
        pltpu.make_async_copy(t_hbm.at[idxc.at[0, 0]], rows.at[b], gsem).wait()

    def scale(ib, b):
        @plsc.parallel_loop(0, CHUNK, step=1, unroll=8)
        def _scale(e):
            a = plsc.bitcast(
                plsc.load_gather(idxc.at[ib, 2], [jnp.full((16,), e, jnp.int32)]),
                jnp.float32)
            for j in range(F // 16):
                rows[b, e, pl.ds(j * 16, 16)] = rows[b, e, pl.ds(j * 16, 16)] * a

    def handle(ch, b, last):
        wait_gather(b)
        if not last:
            wait_idx(ch + 1, 1 - b)
            start_gather(1 - b)
        scale(b, b)
        pltpu.sync_copy(rows.at[b], acc.at[idxc.at[b, 1]], add=True)
        if not last:
            start_idx(jnp.minimum(ch + 2, NCHUNK - 1), b)

    pltpu.sync_copy(idx_hbm.at[wid, 0], idxc.at[0])
    start_idx(1, 1)
    start_gather(0)

    def pipe_body(i, carry):
        for b in range(2):
            handle(2 * i + b, b, False)
        return carry
    lax.fori_loop(0, (NCHUNK - 1) // 2, pipe_body, 0)
    wait_idx(NCHUNK - 1, 1)
    handle(NCHUNK - 1, 0, True)

    plsc.subcore_barrier()
    sl = pl.ds(sid * ROWS_PT, ROWS_PT)
    pltpu.sync_copy(acc.at[sl], out_hbm.at[cid, sl])


_sc_scatter = functools.partial(
    pl.kernel,
    mesh=plsc.VectorSubcoreMesh(core_axis_name="c", subcore_axis_name="s"),
    out_type=jax.ShapeDtypeStruct((NC, NPAD, F), jnp.float32),
    scratch_types=[
        pltpu.VMEM_SHARED((NPAD, F), jnp.float32),
        pltpu.VMEM((2, 3, CHUNK), jnp.int32),
        pltpu.VMEM((2, CHUNK, F), jnp.float32),
        pltpu.SemaphoreType.DMA,
        pltpu.SemaphoreType.DMA,
    ],
    compiler_params=pltpu.CompilerParams(needs_layout_passes=False),
)(_sc_scatter_kernel)


def _msg_body(h_ref, w1_ref, b1_ref, w2_ref, b2_ref, t_ref):
    a = jnp.maximum(
        jnp.dot(h_ref[...], w1_ref[...], preferred_element_type=jnp.float32)
        + b1_ref[...], 0.0)
    t_ref[...] = jnp.dot(a, w2_ref[...], preferred_element_type=jnp.float32) \
        + b2_ref[...]


def _msg_tc(h, w1, b1, w2, b2):
    return pl.pallas_call(
        _msg_body,
        out_shape=jax.ShapeDtypeStruct((N, F), jnp.float32),
    )(h, w1, b1, w2, b2)


def _update_h(h_ref, g0_ref, g1_ref, ew_ref, w1h_ref, w1a_ref, b1_ref,
              w2_ref, b2_ref, gam_ref, bet_ref):
    agg = (g0_ref[:N, :] + g1_ref[:N, :]) * ew_ref[...]
    a = jnp.maximum(
        jnp.dot(h_ref[...], w1h_ref[...], preferred_element_type=jnp.float32)
        + jnp.dot(agg, w1a_ref[...], preferred_element_type=jnp.float32)
        + b1_ref[...], 0.0)
    u = jnp.dot(a, w2_ref[...], preferred_element_type=jnp.float32) + b2_ref[...]
    mean = jnp.mean(u, axis=0, keepdims=True)
    var = jnp.mean((u - mean) * (u - mean), axis=0, keepdims=True)
    hn = gam_ref[...] * (u - mean) / jnp.sqrt(var + 1e-5) + bet_ref[...]
    return jnp.maximum(hn, 0.0)


def _update_msg_body(h_ref, g0_ref, g1_ref, ew_ref, w1h_ref, w1a_ref, b1_ref,
                     w2_ref, b2_ref, gam_ref, bet_ref,
                     mw1_ref, mb1_ref, mw2_ref, mb2_ref, h_out, t_out):
    hn = _update_h(h_ref, g0_ref, g1_ref, ew_ref, w1h_ref, w1a_ref, b1_ref,
                   w2_ref, b2_ref, gam_ref, bet_ref)
    h_out[...] = hn
    a = jnp.maximum(
        jnp.dot(hn, mw1_ref[...], preferred_element_type=jnp.float32)
        + mb1_ref[...], 0.0)
    t_out[...] = jnp.dot(a, mw2_ref[...], preferred_element_type=jnp.float32) \
        + mb2_ref[...]


def _update_msg_tc(h, g0, g1, ew, w1h, w1a, b1, w2, b2, gam, bet,
                   mw1, mb1, mw2, mb2):
    return pl.pallas_call(
        _update_msg_body,
        out_shape=(jax.ShapeDtypeStruct((N, F), jnp.float32),
                   jax.ShapeDtypeStruct((N, F), jnp.float32)),
    )(h, g0, g1, ew, w1h, w1a, b1, w2, b2, gam, bet, mw1, mb1, mw2, mb2)


def _update_pool_body(h_ref, g0_ref, g1_ref, ew_ref, w1h_ref, w1a_ref, b1_ref,
                      w2_ref, b2_ref, gam_ref, bet_ref,
                      batch_ref, lw_ref, lb_ref, out_ref):
    hn = _update_h(h_ref, g0_ref, g1_ref, ew_ref, w1h_ref, w1a_ref, b1_ref,
                   w2_ref, b2_ref, gam_ref, bet_ref)
    gid = lax.broadcasted_iota(jnp.int32, (N, NUM_GRAPHS), 1)
    onehot = jnp.where(batch_ref[...] == gid, 1.0, 0.0).astype(jnp.float32)
    pooled = lax.dot_general(onehot, hn, (((0,), (0,)), ((), ())),
                             preferred_element_type=jnp.float32)
    counts = lax.dot_general(onehot, jnp.ones((N, 1), jnp.float32),
                             (((0,), (0,)), ((), ())),
                             preferred_element_type=jnp.float32)
    pooled = pooled / jnp.maximum(counts, 1.0)
    out_ref[...] = jnp.dot(pooled, lw_ref[...],
                           preferred_element_type=jnp.float32) + lb_ref[...]


def _update_pool_tc(h, g0, g1, ew, w1h, w1a, b1, w2, b2, gam, bet,
                    batch2d, lw, lb):
    return pl.pallas_call(
        _update_pool_body,
        out_shape=jax.ShapeDtypeStruct((NUM_GRAPHS, lw.shape[1]), jnp.float32),
    )(h, g0, g1, ew, w1h, w1a, b1, w2, b2, gam, bet, batch2d, lw, lb)


def kernel(x, edge_index, batch, edge_attr, params):
    extra = E_PAD - E
    src = jnp.concatenate(
        [edge_index[0], jnp.zeros((extra,), jnp.int32)]).reshape(
            NW, NCHUNK, CHUNK)
    dst = jnp.concatenate(
        [edge_index[1], jnp.full((extra,), NPAD - 1, jnp.int32)]).reshape(
            NW, NCHUNK, CHUNK)
    attr_bits = jax.lax.bitcast_convert_type(
        jnp.concatenate([edge_attr.reshape(E), jnp.zeros((extra,), jnp.float32)]
                        ).reshape(NW, NCHUNK, CHUNK), jnp.int32)
    idx_packed = jnp.stack([src, dst, attr_bits], axis=2)
    acc_zeros = jnp.zeros((ROWS_PT, F), jnp.float32)
    batch2d = batch.reshape(N, 1)

    def upd_args(h, g, p):
        return (h, g[0], g[1], p['edge_W'],
                p['upd_W1'][:F], p['upd_W1'][F:],
                p['upd_b1'].reshape(1, F), p['upd_W2'],
                p['upd_b2'].reshape(1, F),
                p['bn_gamma'].reshape(1, F), p['bn_beta'].reshape(1, F))

    p0, p1, p2 = params['layers']
    h = x
    t = _msg_tc(h, p0['msg_W1'], p0['msg_b1'].reshape(1, F),
                p0['msg_W2'], p0['msg_b2'].reshape(1, F))
    for p, pn in ((p0, p1), (p1, p2)):
        g = _sc_scatter(t, idx_packed, acc_zeros)
        h, t = _update_msg_tc(*upd_args(h, g, p),
                              pn['msg_W1'], pn['msg_b1'].reshape(1, F),
                              pn['msg_W2'], pn['msg_b2'].reshape(1, F))
    g = _sc_scatter(t, idx_packed, acc_zeros)
    return _update_pool_tc(*upd_args(h, g, p2), batch2d,
                           params['lin_W'], params['lin_b'].reshape(1, -1))


# speedup vs baseline: 1.4911x; 1.4840x over previous
import functools

import jax
import jax.numpy as jnp
from jax import lax
from jax.experimental import pallas as pl
from jax.experimental.pallas import tpu as pltpu
from jax.experimental.pallas import tpu_sc as plsc

N = 10000
E = 320000
F = 128
NUM_GRAPHS = 8 * 8

NC = 2
NS = 16
NW = NC * NS
CHUNK = 80
NCHUNK = 125
EPT = NCHUNK * CHUNK
NPAD = 10240
ROWS_PT = NPAD // NS
ZROWS = 80


def _sc_scatter_kernel(t_hbm, idx_hbm, out_hbm, acc, idxc,
                       rows, zbuf, gsem, isem):
    cid = lax.axis_index("c")
    sid = lax.axis_index("s")
    wid = sid * NC + cid

    @plsc.parallel_loop(0, ZROWS, step=1, unroll=8)
    def _zrow(i):
        for j in range(F // 16):
            zbuf[i, pl.ds(j * 16, 16)] = jnp.zeros((16,), jnp.float32)
    for k in range(ROWS_PT // ZROWS):
        pltpu.sync_copy(zbuf, acc.at[pl.ds(sid * ROWS_PT + k * ZROWS, ZROWS)])
    plsc.subcore_barrier()

    def start_idx(i, ib):
        pltpu.async_copy(idx_hbm.at[wid, i], idxc.at[ib], isem)

    def wait_idx(i, ib):
        pltpu.make_async_copy(idx_hbm.at[wid, i], idxc.at[ib], isem).wait()

    def start_gather(b):
        pltpu.async_copy(t_hbm.at[idxc.at[b, 0]], rows.at[b], gsem)

    def wait_gather(b):
        pltpu.make_async_copy(t_hbm.at[idxc.at[0, 0]], rows.at[b], gsem).wait()

    def scale(ib, b):
        @plsc.parallel_loop(0, CHUNK, step=1, unroll=8)
        def _scale(e):
            a = plsc.bitcast(
                plsc.load_gather(idxc.at[ib, 2], [jnp.full((16,), e, jnp.int32)]),
                jnp.float32)
            for j in range(F // 16):
                rows[b, e, pl.ds(j * 16, 16)] = rows[b, e, pl.ds(j * 16, 16)] * a

    def handle(ch, b, last):
        wait_gather(b)
        if not last:
            wait_idx(ch + 1, 1 - b)
            start_gather(1 - b)
        scale(b, b)
        pltpu.sync_copy(rows.at[b], acc.at[idxc.at[b, 1]], add=True)
        if not last:
            start_idx(jnp.minimum(ch + 2, NCHUNK - 1), b)

    pltpu.sync_copy(idx_hbm.at[wid, 0], idxc.at[0])
    start_idx(1, 1)
    start_gather(0)

    def pipe_body(i, carry):
        for b in range(2):
            handle(2 * i + b, b, False)
        return carry
    lax.fori_loop(0, (NCHUNK - 1) // 2, pipe_body, 0)
    wait_idx(NCHUNK - 1, 1)
    handle(NCHUNK - 1, 0, True)

    plsc.subcore_barrier()
    sl = pl.ds(sid * ROWS_PT, ROWS_PT)
    pltpu.sync_copy(acc.at[sl], out_hbm.at[cid, sl])


_sc_scatter = functools.partial(
    pl.kernel,
    mesh=plsc.VectorSubcoreMesh(core_axis_name="c", subcore_axis_name="s"),
    out_type=jax.ShapeDtypeStruct((NC, NPAD, F), jnp.float32),
    scratch_types=[
        pltpu.VMEM_SHARED((NPAD, F), jnp.float32),
        pltpu.VMEM((2, 3, CHUNK), jnp.int32),
        pltpu.VMEM((2, CHUNK, F), jnp.float32),
        pltpu.VMEM((ZROWS, F), jnp.float32),
        pltpu.SemaphoreType.DMA,
        pltpu.SemaphoreType.DMA,
    ],
    compiler_params=pltpu.CompilerParams(needs_layout_passes=False),
)(_sc_scatter_kernel)


def _msg_body(h_ref, w1_ref, b1_ref, w2_ref, b2_ref, t_ref):
    a = jnp.maximum(
        jnp.dot(h_ref[...], w1_ref[...], preferred_element_type=jnp.float32)
        + b1_ref[...], 0.0)
    t_ref[...] = jnp.dot(a, w2_ref[...], preferred_element_type=jnp.float32) \
        + b2_ref[...]


def _msg_tc(h, w1, b1, w2, b2):
    return pl.pallas_call(
        _msg_body,
        out_shape=jax.ShapeDtypeStruct((N, F), jnp.float32),
    )(h, w1, b1, w2, b2)


def _update_h(h_ref, g0_ref, g1_ref, ew_ref, w1h_ref, w1a_ref, b1_ref,
              w2_ref, b2_ref, gam_ref, bet_ref):
    agg = (g0_ref[:N, :] + g1_ref[:N, :]) * ew_ref[...]
    a = jnp.maximum(
        jnp.dot(h_ref[...], w1h_ref[...], preferred_element_type=jnp.float32)
        + jnp.dot(agg, w1a_ref[...], preferred_element_type=jnp.float32)
        + b1_ref[...], 0.0)
    u = jnp.dot(a, w2_ref[...], preferred_element_type=jnp.float32) + b2_ref[...]
    mean = jnp.mean(u, axis=0, keepdims=True)
    var = jnp.mean((u - mean) * (u - mean), axis=0, keepdims=True)
    hn = gam_ref[...] * (u - mean) / jnp.sqrt(var + 1e-5) + bet_ref[...]
    return jnp.maximum(hn, 0.0)


def _update_msg_body(h_ref, g0_ref, g1_ref, ew_ref, w1h_ref, w1a_ref, b1_ref,
                     w2_ref, b2_ref, gam_ref, bet_ref,
                     mw1_ref, mb1_ref, mw2_ref, mb2_ref, h_out, t_out):
    hn = _update_h(h_ref, g0_ref, g1_ref, ew_ref, w1h_ref, w1a_ref, b1_ref,
                   w2_ref, b2_ref, gam_ref, bet_ref)
    h_out[...] = hn
    a = jnp.maximum(
        jnp.dot(hn, mw1_ref[...], preferred_element_type=jnp.float32)
        + mb1_ref[...], 0.0)
    t_out[...] = jnp.dot(a, mw2_ref[...], preferred_element_type=jnp.float32) \
        + mb2_ref[...]


def _update_msg_tc(h, g0, g1, ew, w1h, w1a, b1, w2, b2, gam, bet,
                   mw1, mb1, mw2, mb2):
    return pl.pallas_call(
        _update_msg_body,
        out_shape=(jax.ShapeDtypeStruct((N, F), jnp.float32),
                   jax.ShapeDtypeStruct((N, F), jnp.float32)),
    )(h, g0, g1, ew, w1h, w1a, b1, w2, b2, gam, bet, mw1, mb1, mw2, mb2)


def _update_pool_body(h_ref, g0_ref, g1_ref, ew_ref, w1h_ref, w1a_ref, b1_ref,
                      w2_ref, b2_ref, gam_ref, bet_ref,
                      batch_ref, lw_ref, lb_ref, out_ref):
    hn = _update_h(h_ref, g0_ref, g1_ref, ew_ref, w1h_ref, w1a_ref, b1_ref,
                   w2_ref, b2_ref, gam_ref, bet_ref)
    gid = lax.broadcasted_iota(jnp.int32, (N, NUM_GRAPHS), 1)
    onehot = jnp.where(batch_ref[...] == gid, 1.0, 0.0).astype(jnp.float32)
    pooled = lax.dot_general(onehot, hn, (((0,), (0,)), ((), ())),
                             preferred_element_type=jnp.float32)
    counts = lax.dot_general(onehot, jnp.ones((N, 1), jnp.float32),
                             (((0,), (0,)), ((), ())),
                             preferred_element_type=jnp.float32)
    pooled = pooled / jnp.maximum(counts, 1.0)
    out_ref[...] = jnp.dot(pooled, lw_ref[...],
                           preferred_element_type=jnp.float32) + lb_ref[...]


def _update_pool_tc(h, g0, g1, ew, w1h, w1a, b1, w2, b2, gam, bet,
                    batch2d, lw, lb):
    return pl.pallas_call(
        _update_pool_body,
        out_shape=jax.ShapeDtypeStruct((NUM_GRAPHS, lw.shape[1]), jnp.float32),
    )(h, g0, g1, ew, w1h, w1a, b1, w2, b2, gam, bet, batch2d, lw, lb)


def kernel(x, edge_index, batch, edge_attr, params):
    src = edge_index[0].reshape(NW, NCHUNK, CHUNK)
    dst = edge_index[1].reshape(NW, NCHUNK, CHUNK)
    attr_bits = jax.lax.bitcast_convert_type(
        edge_attr.reshape(NW, NCHUNK, CHUNK), jnp.int32)
    idx_packed = jnp.stack([src, dst, attr_bits], axis=2)
    batch2d = batch.reshape(N, 1)

    def upd_args(h, g, p):
        return (h, g[0], g[1], p['edge_W'],
                p['upd_W1'][:F], p['upd_W1'][F:],
                p['upd_b1'].reshape(1, F), p['upd_W2'],
                p['upd_b2'].reshape(1, F),
                p['bn_gamma'].reshape(1, F), p['bn_beta'].reshape(1, F))

    p0, p1, p2 = params['layers']
    h = x
    t = _msg_tc(h, p0['msg_W1'], p0['msg_b1'].reshape(1, F),
                p0['msg_W2'], p0['msg_b2'].reshape(1, F))
    for p, pn in ((p0, p1), (p1, p2)):
        g = _sc_scatter(t, idx_packed)
        h, t = _update_msg_tc(*upd_args(h, g, p),
                              pn['msg_W1'], pn['msg_b1'].reshape(1, F),
                              pn['msg_W2'], pn['msg_b2'].reshape(1, F))
    g = _sc_scatter(t, idx_packed)
    return _update_pool_tc(*upd_args(h, g, p2), batch2d,
                           params['lin_W'], params['lin_b'].reshape(1, -1))
